# baseline (device time: 11608 ns/iter reference)
import functools

import jax
import jax.numpy as jnp
from jax import lax
from jax.experimental import pallas as pl
from jax.experimental.pallas import tpu as pltpu

M = 512
S = 320
F = M - S
C = 32
K = S // C
J = F // C


def kernel(x):
    m_per, n = x.shape
    assert m_per == M and n == 2 * M

    def body(x_hbm, out_hbm, vin, vdiag, send_buf, recv_buf, diag_bf,
             in_sems, diag_in_sem, out_sems,
             sx_send, sx_recv, sy_send, sy_recv):
        my_y = lax.axis_index("y")
        yn = 1 - my_y

        def row(j, py):
            return jnp.where(py == 0, j * C, M - (j + 1) * C)

        def run(px):
            pp = 1 - px
            my_row = functools.partial(row, py=my_y)
            nb_row = functools.partial(row, py=yn)
            lo = my_y * F

            barrier_sem = pltpu.get_barrier_semaphore()
            for dev in ((pp, my_y), (px, yn)):
                pl.semaphore_signal(
                    barrier_sem, inc=1,
                    device_id=dev, device_id_type=pl.DeviceIdType.MESH,
                )

            def in_fetch(j):
                r = my_row(j)
                return pltpu.make_async_copy(
                    x_hbm.at[pl.ds(r, C), pl.ds(pp * M, M)],
                    vin.at[pl.ds(r, C), :],
                    in_sems.at[j],
                )

            def x_send(j):
                r = my_row(j)
                return pltpu.make_async_remote_copy(
                    src_ref=send_buf.at[pl.ds(r, C), :],
                    dst_ref=recv_buf.at[pl.ds(r, C), :],
                    send_sem=sx_send.at[j],
                    recv_sem=sx_recv.at[j],
                    device_id=(pp, my_y),
                    device_id_type=pl.DeviceIdType.MESH,
                )

            def x_recv(j):
                r = my_row(j)
                return pltpu.make_async_remote_copy(
                    src_ref=send_buf.at[pl.ds(r, C), :],
                    dst_ref=recv_buf.at[pl.ds(r, C), :],
                    send_sem=sx_send.at[j],
                    recv_sem=sx_recv.at[j],
                    device_id=(pp, my_y),
                    device_id_type=pl.DeviceIdType.MESH,
                )

            def y_fwd(j):
                r = my_row(j)
                return pltpu.make_async_remote_copy(
                    src_ref=recv_buf.at[pl.ds(r, C), :],
                    dst_ref=out_hbm.at[pl.ds(pp * M + r, C), :],
                    send_sem=sy_send.at[j],
                    recv_sem=sy_recv.at[j],
                    device_id=(px, yn),
                    device_id_type=pl.DeviceIdType.MESH,
                )

            def y_recv(j):
                r = nb_row(j)
                return pltpu.make_async_remote_copy(
                    src_ref=recv_buf.at[pl.ds(r, C), :],
                    dst_ref=out_hbm.at[pl.ds(pp * M + r, C), :],
                    send_sem=sy_send.at[j],
                    recv_sem=sy_recv.at[j],
                    device_id=(px, yn),
                    device_id_type=pl.DeviceIdType.MESH,
                )

            for j in range(K):
                in_fetch(j).start()
            diag_fetch = pltpu.make_async_copy(
                x_hbm.at[:, pl.ds(px * M, M)], vdiag, diag_in_sem,
            )
            diag_fetch.start()

            pl.semaphore_wait(barrier_sem, 2)

            for j in range(K):
                r = my_row(j)
                in_fetch(j).wait()
                send_buf[pl.ds(r, C), :] = vin[pl.ds(r, C), :].astype(
                    jnp.bfloat16
                )
                x_send(j).start()

            diag_fetch.wait()
            diag_bf[...] = vdiag[...].astype(jnp.bfloat16)
            diag_out = pltpu.make_async_copy(
                diag_bf, out_hbm.at[pl.ds(px * M, M), :], out_sems.at[0],
            )
            diag_out.start()

            for j in range(J):
                x_recv(j).wait_recv()
                y_fwd(j).start()
            for j in range(J, K):
                x_recv(j).wait_recv()

            direct_out = pltpu.make_async_copy(
                recv_buf.at[pl.ds(lo, S), :],
                out_hbm.at[pl.ds(pp * M + lo, S), :],
                out_sems.at[1],
            )
            direct_out.start()

            for j in range(J):
                y_recv(j).wait_recv()
            diag_out.wait()
            direct_out.wait()
            for j in range(K):
                x_send(j).wait_send()
            for j in range(J):
                y_fwd(j).wait_send()

        pl.when(lax.axis_index("x") == 0)(functools.partial(run, 0))
        pl.when(lax.axis_index("x") == 1)(functools.partial(run, 1))

    return pl.pallas_call(
        body,
        out_shape=jax.ShapeDtypeStruct((2 * M, M), jnp.bfloat16),
        in_specs=[pl.BlockSpec(memory_space=pl.ANY)],
        out_specs=pl.BlockSpec(memory_space=pl.ANY),
        scratch_shapes=[
            pltpu.VMEM((M, M), jnp.float32),
            pltpu.VMEM((M, M), jnp.float32),
            pltpu.VMEM((M, M), jnp.bfloat16),
            pltpu.VMEM((M, M), jnp.bfloat16),
            pltpu.VMEM((M, M), jnp.bfloat16),
            pltpu.SemaphoreType.DMA((K,)),
            pltpu.SemaphoreType.DMA,
            pltpu.SemaphoreType.DMA((2,)),
            pltpu.SemaphoreType.DMA((K,)),
            pltpu.SemaphoreType.DMA((K,)),
            pltpu.SemaphoreType.DMA((J,)),
            pltpu.SemaphoreType.DMA((J,)),
        ],
        compiler_params=pltpu.CompilerParams(collective_id=0),
    )(x)


# device time: 11555 ns/iter; 1.0046x vs baseline; 1.0046x over previous
import functools

import jax
import jax.numpy as jnp
from jax import lax
from jax.experimental import pallas as pl
from jax.experimental.pallas import tpu as pltpu

M = 512
S = 320
F = M - S
C = 32
K = S // C
J = F // C


def kernel(x):
    m_per, n = x.shape
    assert m_per == M and n == 2 * M
    x = x.astype(jnp.bfloat16)

    def body(x_ref, out_ref, sx_send, sx_recv, sy_send, sy_recv):
        my_y = lax.axis_index("y")
        yn = 1 - my_y

        def row(j, py):
            return jnp.where(py == 0, j * C, M - (j + 1) * C)

        def run(px):
            pp = 1 - px
            my_row = functools.partial(row, py=my_y)
            nb_row = functools.partial(row, py=yn)

            barrier_sem = pltpu.get_barrier_semaphore()
            for dev in ((pp, my_y), (px, yn)):
                pl.semaphore_signal(
                    barrier_sem, inc=1,
                    device_id=dev, device_id_type=pl.DeviceIdType.MESH,
                )

            def x_send(j):
                r = my_row(j)
                return pltpu.make_async_remote_copy(
                    src_ref=x_ref.at[pl.ds(r, C), pl.ds(pp * M, M)],
                    dst_ref=out_ref.at[pl.ds(px * M + r, C), :],
                    send_sem=sx_send.at[j],
                    recv_sem=sx_recv.at[j],
                    device_id=(pp, my_y),
                    device_id_type=pl.DeviceIdType.MESH,
                )

            def x_recv(j):
                r = my_row(j)
                return pltpu.make_async_remote_copy(
                    src_ref=x_ref.at[pl.ds(r, C), pl.ds(pp * M, M)],
                    dst_ref=out_ref.at[pl.ds(pp * M + r, C), :],
                    send_sem=sx_send.at[j],
                    recv_sem=sx_recv.at[j],
                    device_id=(pp, my_y),
                    device_id_type=pl.DeviceIdType.MESH,
                )

            def y_fwd(j):
                r = my_row(j)
                return pltpu.make_async_remote_copy(
                    src_ref=out_ref.at[pl.ds(pp * M + r, C), :],
                    dst_ref=out_ref.at[pl.ds(pp * M + r, C), :],
                    send_sem=sy_send.at[j],
                    recv_sem=sy_recv.at[j],
                    device_id=(px, yn),
                    device_id_type=pl.DeviceIdType.MESH,
                )

            def y_recv(j):
                r = nb_row(j)
                return pltpu.make_async_remote_copy(
                    src_ref=out_ref.at[pl.ds(pp * M + r, C), :],
                    dst_ref=out_ref.at[pl.ds(pp * M + r, C), :],
                    send_sem=sy_send.at[j],
                    recv_sem=sy_recv.at[j],
                    device_id=(px, yn),
                    device_id_type=pl.DeviceIdType.MESH,
                )

            pl.semaphore_wait(barrier_sem, 2)

            for j in range(K):
                x_send(j).start()

            out_ref[pl.ds(px * M, M), :] = x_ref[:, px * M:(px + 1) * M]

            for j in range(J):
                x_recv(j).wait_recv()
                y_fwd(j).start()
            for j in range(J, K):
                x_recv(j).wait_recv()

            for j in range(J):
                y_recv(j).wait_recv()
            for j in range(K):
                x_send(j).wait_send()
            for j in range(J):
                y_fwd(j).wait_send()

        pl.when(lax.axis_index("x") == 0)(functools.partial(run, 0))
        pl.when(lax.axis_index("x") == 1)(functools.partial(run, 1))

    return pl.pallas_call(
        body,
        out_shape=jax.ShapeDtypeStruct((2 * M, M), jnp.bfloat16),
        in_specs=[pl.BlockSpec(memory_space=pltpu.VMEM)],
        out_specs=pl.BlockSpec(memory_space=pltpu.VMEM),
        scratch_shapes=[
            pltpu.SemaphoreType.DMA((K,)),
            pltpu.SemaphoreType.DMA((K,)),
            pltpu.SemaphoreType.DMA((J,)),
            pltpu.SemaphoreType.DMA((J,)),
        ],
        compiler_params=pltpu.CompilerParams(collective_id=0),
    )(x)


# device time: 11349 ns/iter; 1.0228x vs baseline; 1.0182x over previous
import functools

import jax
import jax.numpy as jnp
from jax import lax
from jax.experimental import pallas as pl
from jax.experimental.pallas import tpu as pltpu

M = 512
S = 320
F = M - S
C = 32
K = S // C
J = F // C


def kernel(x):
    m_per, n = x.shape
    assert m_per == M and n == 2 * M

    def body(x_ref, out_ref, send_buf, sx_send, sx_recv, sy_send, sy_recv):
        my_y = lax.axis_index("y")
        yn = 1 - my_y

        def row(j, py):
            return jnp.where(py == 0, j * C, M - (j + 1) * C)

        def run(px):
            pp = 1 - px
            my_row = functools.partial(row, py=my_y)
            nb_row = functools.partial(row, py=yn)

            barrier_sem = pltpu.get_barrier_semaphore()
            for dev in ((pp, my_y), (px, yn)):
                pl.semaphore_signal(
                    barrier_sem, inc=1,
                    device_id=dev, device_id_type=pl.DeviceIdType.MESH,
                )

            def x_send(j):
                r = my_row(j)
                return pltpu.make_async_remote_copy(
                    src_ref=send_buf.at[pl.ds(r, C), :],
                    dst_ref=out_ref.at[pl.ds(px * M + r, C), :],
                    send_sem=sx_send.at[j],
                    recv_sem=sx_recv.at[j],
                    device_id=(pp, my_y),
                    device_id_type=pl.DeviceIdType.MESH,
                )

            def x_recv(j):
                r = my_row(j)
                return pltpu.make_async_remote_copy(
                    src_ref=send_buf.at[pl.ds(r, C), :],
                    dst_ref=out_ref.at[pl.ds(pp * M + r, C), :],
                    send_sem=sx_send.at[j],
                    recv_sem=sx_recv.at[j],
                    device_id=(pp, my_y),
                    device_id_type=pl.DeviceIdType.MESH,
                )

            def y_fwd(j):
                r = my_row(j)
                return pltpu.make_async_remote_copy(
                    src_ref=out_ref.at[pl.ds(pp * M + r, C), :],
                    dst_ref=out_ref.at[pl.ds(pp * M + r, C), :],
                    send_sem=sy_send.at[j],
                    recv_sem=sy_recv.at[j],
                    device_id=(px, yn),
                    device_id_type=pl.DeviceIdType.MESH,
                )

            def y_recv(j):
                r = nb_row(j)
                return pltpu.make_async_remote_copy(
                    src_ref=out_ref.at[pl.ds(pp * M + r, C), :],
                    dst_ref=out_ref.at[pl.ds(pp * M + r, C), :],
                    send_sem=sy_send.at[j],
                    recv_sem=sy_recv.at[j],
                    device_id=(px, yn),
                    device_id_type=pl.DeviceIdType.MESH,
                )

            pl.semaphore_wait(barrier_sem, 2)

            for j in range(K):
                r = my_row(j)
                send_buf[pl.ds(r, C), :] = (
                    x_ref[pl.ds(r, C), pp * M:(pp + 1) * M]
                    .astype(jnp.bfloat16)
                )
                x_send(j).start()

            out_ref[pl.ds(px * M, M), :] = (
                x_ref[:, px * M:(px + 1) * M].astype(jnp.bfloat16)
            )

            for j in range(J):
                x_recv(j).wait_recv()
                y_fwd(j).start()
            for j in range(J, K):
                x_recv(j).wait_recv()

            for j in range(J):
                y_recv(j).wait_recv()
            for j in range(K):
                x_send(j).wait_send()
            for j in range(J):
                y_fwd(j).wait_send()

        pl.when(lax.axis_index("x") == 0)(functools.partial(run, 0))
        pl.when(lax.axis_index("x") == 1)(functools.partial(run, 1))

    return pl.pallas_call(
        body,
        out_shape=jax.ShapeDtypeStruct((2 * M, M), jnp.bfloat16),
        in_specs=[pl.BlockSpec(memory_space=pltpu.VMEM)],
        out_specs=pl.BlockSpec(memory_space=pltpu.VMEM),
        scratch_shapes=[
            pltpu.VMEM((M, M), jnp.bfloat16),
            pltpu.SemaphoreType.DMA((K,)),
            pltpu.SemaphoreType.DMA((K,)),
            pltpu.SemaphoreType.DMA((J,)),
            pltpu.SemaphoreType.DMA((J,)),
        ],
        compiler_params=pltpu.CompilerParams(collective_id=0),
    )(x)
